# initial kernel scaffold (unmeasured)
import jax
import jax.numpy as jnp
from jax import lax
from jax.experimental import pallas as pl
from jax.experimental.pallas import tpu as pltpu

B, SQ, H, D = 4, 32, 8, 128
KV_PER_SHARD = 4096
CK = 512
N_CHUNKS = KV_PER_SHARD // CK
SCALE = D ** -0.5


def kernel(Q, K, V):
    def body(q_ref, k_ref, v_ref, out_ref,
             qbuf, kbuf, vbuf, accl, sendbuf, recvbuf,
             ksem, vsem, send_sems, recv_sems):
        my_x = lax.axis_index("x")
        my_y = lax.axis_index("y")
        my_z = lax.axis_index("z")

        qbuf[...] = (q_ref[...] * SCALE).astype(jnp.bfloat16)
        accl[...] = jnp.zeros_like(accl)

        def chunk_body(c, _):
            cp_k = pltpu.make_async_copy(
                k_ref.at[:, pl.ds(c * CK, CK), :, :], kbuf, ksem)
            cp_v = pltpu.make_async_copy(
                v_ref.at[:, pl.ds(c * CK, CK), :, :], vbuf, vsem)
            cp_k.start()
            cp_v.start()
            cp_k.wait()
            cp_v.wait()
            for b in range(B):
                for h in range(H):
                    kb = kbuf[b, :, h, :].astype(jnp.bfloat16)
                    qb = qbuf[b, :, h, :]
                    s = lax.dot_general(
                        qb, kb, (((1,), (1,)), ((), ())),
                        preferred_element_type=jnp.float32)
                    p = jnp.exp(s)
                    lsum = jnp.sum(p, axis=1, keepdims=True)
                    accl[1, b, h] = accl[1, b, h] + jnp.broadcast_to(
                        lsum, (SQ, D))
                    vb = vbuf[b, :, h, :].astype(jnp.bfloat16)
                    pv = lax.dot_general(
                        p.astype(jnp.bfloat16), vb,
                        (((1,), (0,)), ((), ())),
                        preferred_element_type=jnp.float32)
                    accl[0, b, h] = accl[0, b, h] + pv
            return 0

        lax.fori_loop(0, N_CHUNKS, chunk_body, 0)

        sendbuf[...] = accl[...].astype(jnp.bfloat16)
        for phase in range(2):
            peer_z = jnp.bitwise_xor(my_z, 1 << phase)
            rdma = pltpu.make_async_remote_copy(
                src_ref=sendbuf,
                dst_ref=recvbuf.at[phase],
                send_sem=send_sems.at[phase],
                recv_sem=recv_sems.at[phase],
                device_id=(my_x, my_y, peer_z),
                device_id_type=pl.DeviceIdType.MESH,
            )
            rdma.start()
            rdma.wait()
            accl[...] = accl[...] + recvbuf[phase].astype(jnp.float32)
            if phase == 0:
                sendbuf[...] = accl[...].astype(jnp.bfloat16)

        for b in range(B):
            for h in range(H):
                out_ref[b, :, h, :] = accl[0, b, h] / accl[1, b, h]

    return pl.pallas_call(
        body,
        out_shape=jax.ShapeDtypeStruct((B, SQ, H, D), jnp.float32),
        in_specs=[
            pl.BlockSpec(memory_space=pltpu.VMEM),
            pl.BlockSpec(memory_space=pltpu.ANY),
            pl.BlockSpec(memory_space=pltpu.ANY),
        ],
        out_specs=pl.BlockSpec(memory_space=pltpu.VMEM),
        scratch_shapes=[
            pltpu.VMEM((B, SQ, H, D), jnp.bfloat16),
            pltpu.VMEM((B, CK, H, D), jnp.float32),
            pltpu.VMEM((B, CK, H, D), jnp.float32),
            pltpu.VMEM((2, B, H, SQ, D), jnp.float32),
            pltpu.VMEM((2, B, H, SQ, D), jnp.bfloat16),
            pltpu.VMEM((2, 2, B, H, SQ, D), jnp.bfloat16),
            pltpu.SemaphoreType.DMA,
            pltpu.SemaphoreType.DMA,
            pltpu.SemaphoreType.DMA((2,)),
            pltpu.SemaphoreType.DMA((2,)),
        ],
        compiler_params=pltpu.CompilerParams(collective_id=0),
    )(Q, K, V)


# baseline (device time: 219320 ns/iter reference)
import jax
import jax.numpy as jnp
from jax import lax
from jax.experimental import pallas as pl
from jax.experimental.pallas import tpu as pltpu

B, SQ, H, D = 4, 32, 8, 128
KV_PER_SHARD = 4096
CK = 512
N_CHUNKS = KV_PER_SHARD // CK
SCALE = D ** -0.5


def kernel(Q, K, V):
    def body(q_ref, k_ref, v_ref, out_ref,
             qbuf, kbuf, vbuf, accl, sendbuf, recvbuf,
             ksem, vsem, send_sems, recv_sems):
        my_x = lax.axis_index("x")
        my_y = lax.axis_index("y")
        my_z = lax.axis_index("z")

        qbuf[...] = (q_ref[...] * SCALE).astype(jnp.bfloat16)
        accl[...] = jnp.zeros_like(accl)

        def chunk_body(c, _):
            cp_k = pltpu.make_async_copy(
                k_ref.at[:, pl.ds(c * CK, CK), :, :], kbuf, ksem)
            cp_v = pltpu.make_async_copy(
                v_ref.at[:, pl.ds(c * CK, CK), :, :], vbuf, vsem)
            cp_k.start()
            cp_v.start()
            cp_k.wait()
            cp_v.wait()
            for b in range(B):
                for h in range(H):
                    kb = kbuf[b, :, h, :].astype(jnp.bfloat16)
                    qb = qbuf[b, :, h, :]
                    s = lax.dot_general(
                        qb, kb, (((1,), (1,)), ((), ())),
                        preferred_element_type=jnp.float32)
                    p = jnp.exp(s)
                    lsum = jnp.sum(p, axis=1, keepdims=True)
                    accl[1, b, h] = accl[1, b, h] + jnp.broadcast_to(
                        lsum, (SQ, D))
                    vb = vbuf[b, :, h, :].astype(jnp.bfloat16)
                    pv = lax.dot_general(
                        p.astype(jnp.bfloat16), vb,
                        (((1,), (0,)), ((), ())),
                        preferred_element_type=jnp.float32)
                    accl[0, b, h] = accl[0, b, h] + pv
            return 0

        lax.fori_loop(0, N_CHUNKS, chunk_body, 0)

        sendbuf[...] = accl[...].astype(jnp.bfloat16)
        for phase in range(2):
            peer_z = jnp.bitwise_xor(my_z, 1 << phase)
            rdma = pltpu.make_async_remote_copy(
                src_ref=sendbuf,
                dst_ref=recvbuf.at[phase],
                send_sem=send_sems.at[phase],
                recv_sem=recv_sems.at[phase],
                device_id=(my_x, my_y, peer_z),
                device_id_type=pl.DeviceIdType.MESH,
            )
            rdma.start()
            rdma.wait()
            accl[...] = accl[...] + recvbuf[phase].astype(jnp.float32)
            if phase == 0:
                sendbuf[...] = accl[...].astype(jnp.bfloat16)

        for b in range(B):
            for h in range(H):
                out_ref[b, :, h, :] = accl[0, b, h] / accl[1, b, h]

    return pl.pallas_call(
        body,
        out_shape=jax.ShapeDtypeStruct((B, SQ, H, D), jnp.float32),
        in_specs=[
            pl.BlockSpec(memory_space=pltpu.VMEM),
            pl.BlockSpec(memory_space=pl.ANY),
            pl.BlockSpec(memory_space=pl.ANY),
        ],
        out_specs=pl.BlockSpec(memory_space=pltpu.VMEM),
        scratch_shapes=[
            pltpu.VMEM((B, SQ, H, D), jnp.bfloat16),
            pltpu.VMEM((B, CK, H, D), jnp.float32),
            pltpu.VMEM((B, CK, H, D), jnp.float32),
            pltpu.VMEM((2, B, H, SQ, D), jnp.float32),
            pltpu.VMEM((2, B, H, SQ, D), jnp.bfloat16),
            pltpu.VMEM((2, 2, B, H, SQ, D), jnp.bfloat16),
            pltpu.SemaphoreType.DMA,
            pltpu.SemaphoreType.DMA,
            pltpu.SemaphoreType.DMA((2,)),
            pltpu.SemaphoreType.DMA((2,)),
        ],
    )(Q, K, V)


# device time: 65534 ns/iter; 3.3467x vs baseline; 3.3467x over previous
import jax
import jax.numpy as jnp
from jax import lax
from jax.experimental import pallas as pl
from jax.experimental.pallas import tpu as pltpu

B, SQ, H, D = 4, 32, 8, 128
KV_PER_SHARD = 4096
CK = 1024
N_CHUNKS = KV_PER_SHARD // CK
SCALE = D ** -0.5


def kernel(Q, K, V):
    def body(q_ref, k_ref, v_ref, out_ref,
             qbuf, kbuf, vbuf, accl, sendbuf, recvbuf, obuf,
             ksem, vsem, send_sems, recv_sems, gsend_sems, grecv_sems):
        my_x = lax.axis_index("x")
        my_y = lax.axis_index("y")
        my_z = lax.axis_index("z")
        my_b = 2 * my_x + my_y

        qbuf[...] = (q_ref[my_b] * SCALE).astype(jnp.bfloat16)
        accl[...] = jnp.zeros_like(accl)

        def start_dma(c, slot):
            cp_k = pltpu.make_async_copy(
                k_ref.at[my_b, pl.ds(c * CK, CK), :, :], kbuf.at[slot],
                ksem.at[slot])
            cp_v = pltpu.make_async_copy(
                v_ref.at[my_b, pl.ds(c * CK, CK), :, :], vbuf.at[slot],
                vsem.at[slot])
            cp_k.start()
            cp_v.start()
            return cp_k, cp_v

        start_dma(0, 0)
        for c in range(N_CHUNKS):
            slot = c % 2
            if c + 1 < N_CHUNKS:
                start_dma(c + 1, (c + 1) % 2)
            pltpu.make_async_copy(
                k_ref.at[my_b, pl.ds(c * CK, CK), :, :], kbuf.at[slot],
                ksem.at[slot]).wait()
            pltpu.make_async_copy(
                v_ref.at[my_b, pl.ds(c * CK, CK), :, :], vbuf.at[slot],
                vsem.at[slot]).wait()
            for h in range(H):
                kb = kbuf[slot, :, h, :].astype(jnp.bfloat16)
                qb = qbuf[:, h, :]
                s = lax.dot_general(
                    qb, kb, (((1,), (1,)), ((), ())),
                    preferred_element_type=jnp.float32)
                p = jnp.exp(s)
                lsum = jnp.sum(p, axis=1, keepdims=True)
                accl[1, h] = accl[1, h] + jnp.broadcast_to(lsum, (SQ, D))
                vb = vbuf[slot, :, h, :].astype(jnp.bfloat16)
                pv = lax.dot_general(
                    p.astype(jnp.bfloat16), vb,
                    (((1,), (0,)), ((), ())),
                    preferred_element_type=jnp.float32)
                accl[0, h] = accl[0, h] + pv

        sendbuf[...] = accl[...].astype(jnp.bfloat16)
        for phase in range(2):
            peer_z = jnp.bitwise_xor(my_z, 1 << phase)
            rdma = pltpu.make_async_remote_copy(
                src_ref=sendbuf,
                dst_ref=recvbuf.at[phase],
                send_sem=send_sems.at[phase],
                recv_sem=recv_sems.at[phase],
                device_id=(my_x, my_y, peer_z),
                device_id_type=pl.DeviceIdType.MESH,
            )
            rdma.start()
            rdma.wait()
            accl[...] = accl[...] + recvbuf[phase].astype(jnp.float32)
            if phase == 0:
                sendbuf[...] = accl[...].astype(jnp.bfloat16)

        for h in range(H):
            obuf[my_b, :, h, :] = (accl[0, h] / accl[1, h]).astype(
                jnp.bfloat16)

        bx = jnp.bitwise_xor(my_b, 2)
        rdma_x = pltpu.make_async_remote_copy(
            src_ref=obuf.at[my_b],
            dst_ref=obuf.at[my_b],
            send_sem=gsend_sems.at[0],
            recv_sem=grecv_sems.at[0],
            device_id=(1 - my_x, my_y, my_z),
            device_id_type=pl.DeviceIdType.MESH,
        )
        rdma_x.start()
        rdma_x.wait()
        for i, src_b in enumerate((my_b, bx)):
            rdma_y = pltpu.make_async_remote_copy(
                src_ref=obuf.at[src_b],
                dst_ref=obuf.at[src_b],
                send_sem=gsend_sems.at[1 + i],
                recv_sem=grecv_sems.at[1 + i],
                device_id=(my_x, 1 - my_y, my_z),
                device_id_type=pl.DeviceIdType.MESH,
            )
            rdma_y.start()
            rdma_y.wait()

        out_ref[...] = obuf[...].astype(jnp.float32)

    return pl.pallas_call(
        body,
        out_shape=jax.ShapeDtypeStruct((B, SQ, H, D), jnp.float32),
        in_specs=[
            pl.BlockSpec(memory_space=pltpu.VMEM),
            pl.BlockSpec(memory_space=pl.ANY),
            pl.BlockSpec(memory_space=pl.ANY),
        ],
        out_specs=pl.BlockSpec(memory_space=pltpu.VMEM),
        scratch_shapes=[
            pltpu.VMEM((SQ, H, D), jnp.bfloat16),
            pltpu.VMEM((2, CK, H, D), jnp.float32),
            pltpu.VMEM((2, CK, H, D), jnp.float32),
            pltpu.VMEM((2, H, SQ, D), jnp.float32),
            pltpu.VMEM((2, H, SQ, D), jnp.bfloat16),
            pltpu.VMEM((2, 2, H, SQ, D), jnp.bfloat16),
            pltpu.VMEM((B, SQ, H, D), jnp.bfloat16),
            pltpu.SemaphoreType.DMA((2,)),
            pltpu.SemaphoreType.DMA((2,)),
            pltpu.SemaphoreType.DMA((2,)),
            pltpu.SemaphoreType.DMA((2,)),
            pltpu.SemaphoreType.DMA((3,)),
            pltpu.SemaphoreType.DMA((3,)),
        ],
    )(Q, K, V)


# device time: 40338 ns/iter; 5.4371x vs baseline; 1.6246x over previous
import jax
import jax.numpy as jnp
from jax import lax
from jax.experimental import pallas as pl
from jax.experimental.pallas import tpu as pltpu

B, SQ, H, D = 4, 32, 8, 128
KV_PER_SHARD = 4096
CK = 1024
N_CHUNKS = KV_PER_SHARD // CK
SCALE = D ** -0.5


def kernel(Q, K, V):
    def body(q_ref, k_ref, v_ref, out_ref,
             qbuf, kbuf, vbuf, accl, sendbuf, recvbuf, obuf,
             ksem, vsem, send_sems, recv_sems, gsend_sems, grecv_sems):
        my_x = lax.axis_index("x")
        my_y = lax.axis_index("y")
        my_z = lax.axis_index("z")
        my_b = 2 * my_x + my_y

        qbuf[...] = (q_ref[my_b] * SCALE).astype(jnp.bfloat16)
        accl[...] = jnp.zeros_like(accl)

        def chunk_copies(c, slot):
            copies = []
            for h in range(H):
                copies.append(pltpu.make_async_copy(
                    k_ref.at[my_b, pl.ds(c * CK, CK), h, :],
                    kbuf.at[slot, h], ksem.at[slot, h]))
                copies.append(pltpu.make_async_copy(
                    v_ref.at[my_b, pl.ds(c * CK, CK), h, :],
                    vbuf.at[slot, h], vsem.at[slot, h]))
            return copies

        def start_dma(c, slot):
            for cp in chunk_copies(c, slot):
                cp.start()

        start_dma(0, 0)
        for c in range(N_CHUNKS):
            slot = c % 2
            if c + 1 < N_CHUNKS:
                start_dma(c + 1, (c + 1) % 2)
            for cp in chunk_copies(c, slot):
                cp.wait()
            for h in range(H):
                kb = kbuf[slot, h].astype(jnp.bfloat16)
                qb = qbuf[:, h, :]
                s = lax.dot_general(
                    qb, kb, (((1,), (1,)), ((), ())),
                    preferred_element_type=jnp.float32)
                p = jnp.exp(s)
                lsum = jnp.sum(p, axis=1, keepdims=True)
                accl[1, h] = accl[1, h] + jnp.broadcast_to(lsum, (SQ, D))
                vb = vbuf[slot, h].astype(jnp.bfloat16)
                pv = lax.dot_general(
                    p.astype(jnp.bfloat16), vb,
                    (((1,), (0,)), ((), ())),
                    preferred_element_type=jnp.float32)
                accl[0, h] = accl[0, h] + pv

        sendbuf[...] = accl[...].astype(jnp.bfloat16)
        for phase in range(2):
            peer_z = jnp.bitwise_xor(my_z, 1 << phase)
            rdma = pltpu.make_async_remote_copy(
                src_ref=sendbuf,
                dst_ref=recvbuf.at[phase],
                send_sem=send_sems.at[phase],
                recv_sem=recv_sems.at[phase],
                device_id=(my_x, my_y, peer_z),
                device_id_type=pl.DeviceIdType.MESH,
            )
            rdma.start()
            rdma.wait()
            accl[...] = accl[...] + recvbuf[phase].astype(jnp.float32)
            if phase == 0:
                sendbuf[...] = accl[...].astype(jnp.bfloat16)

        for h in range(H):
            obuf[my_b, :, h, :] = (accl[0, h] / accl[1, h]).astype(
                jnp.bfloat16)

        bx = jnp.bitwise_xor(my_b, 2)
        rdma_x = pltpu.make_async_remote_copy(
            src_ref=obuf.at[my_b],
            dst_ref=obuf.at[my_b],
            send_sem=gsend_sems.at[0],
            recv_sem=grecv_sems.at[0],
            device_id=(1 - my_x, my_y, my_z),
            device_id_type=pl.DeviceIdType.MESH,
        )
        rdma_x.start()
        rdma_x.wait()
        for i, src_b in enumerate((my_b, bx)):
            rdma_y = pltpu.make_async_remote_copy(
                src_ref=obuf.at[src_b],
                dst_ref=obuf.at[src_b],
                send_sem=gsend_sems.at[1 + i],
                recv_sem=grecv_sems.at[1 + i],
                device_id=(my_x, 1 - my_y, my_z),
                device_id_type=pl.DeviceIdType.MESH,
            )
            rdma_y.start()
            rdma_y.wait()

        out_ref[...] = obuf[...].astype(jnp.float32)

    return pl.pallas_call(
        body,
        out_shape=jax.ShapeDtypeStruct((B, SQ, H, D), jnp.float32),
        in_specs=[
            pl.BlockSpec(memory_space=pltpu.VMEM),
            pl.BlockSpec(memory_space=pl.ANY),
            pl.BlockSpec(memory_space=pl.ANY),
        ],
        out_specs=pl.BlockSpec(memory_space=pltpu.VMEM),
        scratch_shapes=[
            pltpu.VMEM((SQ, H, D), jnp.bfloat16),
            pltpu.VMEM((2, H, CK, D), jnp.float32),
            pltpu.VMEM((2, H, CK, D), jnp.float32),
            pltpu.VMEM((2, H, SQ, D), jnp.float32),
            pltpu.VMEM((2, H, SQ, D), jnp.bfloat16),
            pltpu.VMEM((2, 2, H, SQ, D), jnp.bfloat16),
            pltpu.VMEM((B, SQ, H, D), jnp.bfloat16),
            pltpu.SemaphoreType.DMA((2, H)),
            pltpu.SemaphoreType.DMA((2, H)),
            pltpu.SemaphoreType.DMA((2,)),
            pltpu.SemaphoreType.DMA((2,)),
            pltpu.SemaphoreType.DMA((3,)),
            pltpu.SemaphoreType.DMA((3,)),
        ],
    )(Q, K, V)


# device time: 33413 ns/iter; 6.5639x vs baseline; 1.2073x over previous
import jax
import jax.numpy as jnp
from jax import lax
from jax.experimental import pallas as pl
from jax.experimental.pallas import tpu as pltpu

B, SQ, H, D = 4, 32, 8, 128
KV_PER_SHARD = 4096
CK = 1024
N_CHUNKS = KV_PER_SHARD // CK
LW = 8
SCALE = D ** -0.5


def kernel(Q, K, V):
    def body(q_ref, k_ref, v_ref, out_ref,
             qbuf, kbuf, vbuf, acc, lbuf, accsend, accr, lr, obuf,
             ksem, vsem, zs_acc, zr_acc, zs_l, zr_l,
             gsend_sems, grecv_sems):
        my_x = lax.axis_index("x")
        my_y = lax.axis_index("y")
        my_z = lax.axis_index("z")
        my_b = 2 * my_x + my_y

        barrier = pltpu.get_barrier_semaphore()
        for dz in (1, 2, 3):
            pl.semaphore_signal(
                barrier, inc=1,
                device_id=(my_x, my_y, jnp.bitwise_xor(my_z, dz)),
                device_id_type=pl.DeviceIdType.MESH)
        pl.semaphore_signal(
            barrier, inc=1, device_id=(1 - my_x, my_y, my_z),
            device_id_type=pl.DeviceIdType.MESH)
        pl.semaphore_signal(
            barrier, inc=1, device_id=(my_x, 1 - my_y, my_z),
            device_id_type=pl.DeviceIdType.MESH)
        pl.semaphore_wait(barrier, 5)

        qbuf[...] = (q_ref[my_b] * SCALE).astype(jnp.bfloat16)
        acc[...] = jnp.zeros_like(acc)
        lbuf[...] = jnp.zeros_like(lbuf)

        def chunk_copies(c, slot):
            copies = []
            for h in range(H):
                copies.append(pltpu.make_async_copy(
                    k_ref.at[my_b, pl.ds(c * CK, CK), h, :],
                    kbuf.at[slot, h], ksem.at[slot, h]))
                copies.append(pltpu.make_async_copy(
                    v_ref.at[my_b, pl.ds(c * CK, CK), h, :],
                    vbuf.at[slot, h], vsem.at[slot, h]))
            return copies

        def start_dma(c, slot):
            for cp in chunk_copies(c, slot):
                cp.start()

        start_dma(0, 0)
        for c in range(N_CHUNKS):
            slot = c % 2
            if c + 1 < N_CHUNKS:
                start_dma(c + 1, (c + 1) % 2)
            for cp in chunk_copies(c, slot):
                cp.wait()
            for h in range(H):
                kb = kbuf[slot, h].astype(jnp.bfloat16)
                qb = qbuf[:, h, :]
                s = lax.dot_general(
                    qb, kb, (((1,), (1,)), ((), ())),
                    preferred_element_type=jnp.float32)
                p = jnp.exp(s)
                lsum = jnp.sum(p, axis=1, keepdims=True)
                lbuf[h] = lbuf[h] + jnp.broadcast_to(lsum, (SQ, LW))
                vb = vbuf[slot, h].astype(jnp.bfloat16)
                pv = lax.dot_general(
                    p.astype(jnp.bfloat16), vb,
                    (((1,), (0,)), ((), ())),
                    preferred_element_type=jnp.float32)
                acc[h] = acc[h] + pv

        accsend[...] = acc[...].astype(jnp.bfloat16)
        rdmas = []
        for j, dz in enumerate((1, 2, 3)):
            peer = (my_x, my_y, jnp.bitwise_xor(my_z, dz))
            r_acc = pltpu.make_async_remote_copy(
                src_ref=accsend, dst_ref=accr.at[j],
                send_sem=zs_acc.at[j], recv_sem=zr_acc.at[j],
                device_id=peer, device_id_type=pl.DeviceIdType.MESH)
            r_l = pltpu.make_async_remote_copy(
                src_ref=lbuf, dst_ref=lr.at[j],
                send_sem=zs_l.at[j], recv_sem=zr_l.at[j],
                device_id=peer, device_id_type=pl.DeviceIdType.MESH)
            r_acc.start()
            r_l.start()
            rdmas.extend((r_acc, r_l))
        for r in rdmas:
            r.wait()
        acc[...] = (acc[...]
                    + accr[0].astype(jnp.float32)
                    + accr[1].astype(jnp.float32)
                    + accr[2].astype(jnp.float32))
        lbuf[...] = lbuf[...] + lr[0] + lr[1] + lr[2]

        for h in range(H):
            linv = jnp.broadcast_to(lbuf[h][:, 0:1], (SQ, D))
            obuf[my_b, :, h, :] = (acc[h] / linv).astype(jnp.bfloat16)

        by = jnp.bitwise_xor(my_b, 1)
        rdma_x1 = pltpu.make_async_remote_copy(
            src_ref=obuf.at[my_b], dst_ref=obuf.at[my_b],
            send_sem=gsend_sems.at[0], recv_sem=grecv_sems.at[0],
            device_id=(1 - my_x, my_y, my_z),
            device_id_type=pl.DeviceIdType.MESH)
        rdma_y1 = pltpu.make_async_remote_copy(
            src_ref=obuf.at[my_b], dst_ref=obuf.at[my_b],
            send_sem=gsend_sems.at[1], recv_sem=grecv_sems.at[1],
            device_id=(my_x, 1 - my_y, my_z),
            device_id_type=pl.DeviceIdType.MESH)
        rdma_x1.start()
        rdma_y1.start()
        rdma_y1.wait()
        rdma_x2 = pltpu.make_async_remote_copy(
            src_ref=obuf.at[by], dst_ref=obuf.at[by],
            send_sem=gsend_sems.at[2], recv_sem=grecv_sems.at[2],
            device_id=(1 - my_x, my_y, my_z),
            device_id_type=pl.DeviceIdType.MESH)
        rdma_x2.start()
        rdma_x1.wait()
        rdma_x2.wait()

        out_ref[...] = obuf[...].astype(jnp.float32)

    return pl.pallas_call(
        body,
        out_shape=jax.ShapeDtypeStruct((B, SQ, H, D), jnp.float32),
        in_specs=[
            pl.BlockSpec(memory_space=pltpu.VMEM),
            pl.BlockSpec(memory_space=pl.ANY),
            pl.BlockSpec(memory_space=pl.ANY),
        ],
        out_specs=pl.BlockSpec(memory_space=pltpu.VMEM),
        scratch_shapes=[
            pltpu.VMEM((SQ, H, D), jnp.bfloat16),
            pltpu.VMEM((2, H, CK, D), jnp.float32),
            pltpu.VMEM((2, H, CK, D), jnp.float32),
            pltpu.VMEM((H, SQ, D), jnp.float32),
            pltpu.VMEM((H, SQ, LW), jnp.float32),
            pltpu.VMEM((H, SQ, D), jnp.bfloat16),
            pltpu.VMEM((3, H, SQ, D), jnp.bfloat16),
            pltpu.VMEM((3, H, SQ, LW), jnp.float32),
            pltpu.VMEM((B, SQ, H, D), jnp.bfloat16),
            pltpu.SemaphoreType.DMA((2, H)),
            pltpu.SemaphoreType.DMA((2, H)),
            pltpu.SemaphoreType.DMA((3,)),
            pltpu.SemaphoreType.DMA((3,)),
            pltpu.SemaphoreType.DMA((3,)),
            pltpu.SemaphoreType.DMA((3,)),
            pltpu.SemaphoreType.DMA((3,)),
            pltpu.SemaphoreType.DMA((3,)),
        ],
        compiler_params=pltpu.CompilerParams(collective_id=0),
    )(Q, K, V)


# device time: 29747 ns/iter; 7.3728x vs baseline; 1.1232x over previous
import jax
import jax.numpy as jnp
from jax import lax
from jax.experimental import pallas as pl
from jax.experimental.pallas import tpu as pltpu

B, SQ, H, D = 4, 32, 8, 128
KV_PER_SHARD = 4096
CK = 1024
N_CHUNKS = KV_PER_SHARD // CK
HG = 4
N_G = H // HG
LW = 8
SCALE = D ** -0.5


def kernel(Q, K, V):
    def body(q_ref, k_ref, v_ref, out_ref,
             qbuf, kbuf, vbuf, acc, lbuf, accsend, accr, lr, obuf,
             ksem, vsem, zs_acc, zr_acc, zs_l, zr_l,
             gsend_sems, grecv_sems):
        my_x = lax.axis_index("x")
        my_y = lax.axis_index("y")
        my_z = lax.axis_index("z")
        my_b = 2 * my_x + my_y

        for dz in (1, 2, 3):
            pl.semaphore_signal(
                pltpu.get_barrier_semaphore(), inc=1,
                device_id=(my_x, my_y, jnp.bitwise_xor(my_z, dz)),
                device_id_type=pl.DeviceIdType.MESH)
        for peer in ((1 - my_x, my_y, my_z), (my_x, 1 - my_y, my_z),
                     (1 - my_x, 1 - my_y, my_z)):
            pl.semaphore_signal(
                pltpu.get_barrier_semaphore(), inc=1, device_id=peer,
                device_id_type=pl.DeviceIdType.MESH)
        pl.semaphore_wait(pltpu.get_barrier_semaphore(), 6)

        qbuf[...] = (q_ref[my_b] * SCALE).astype(jnp.bfloat16)
        acc[...] = jnp.zeros_like(acc)
        lbuf[...] = jnp.zeros_like(lbuf)

        def chunk_copies(t, slot):
            g, c = divmod(t, N_CHUNKS)
            copies = []
            for hl in range(HG):
                h = g * HG + hl
                copies.append(pltpu.make_async_copy(
                    k_ref.at[my_b, pl.ds(c * CK, CK), h, :],
                    kbuf.at[slot, hl], ksem.at[slot, hl]))
                copies.append(pltpu.make_async_copy(
                    v_ref.at[my_b, pl.ds(c * CK, CK), h, :],
                    vbuf.at[slot, hl], vsem.at[slot, hl]))
            return copies

        def start_dma(t, slot):
            for cp in chunk_copies(t, slot):
                cp.start()

        def z_rdmas(g):
            hs = pl.ds(g * HG, HG)
            rdmas = []
            for j, dz in enumerate((1, 2, 3)):
                peer = (my_x, my_y, jnp.bitwise_xor(my_z, dz))
                rdmas.append(pltpu.make_async_remote_copy(
                    src_ref=accsend.at[hs], dst_ref=accr.at[j, hs],
                    send_sem=zs_acc.at[g, j], recv_sem=zr_acc.at[g, j],
                    device_id=peer, device_id_type=pl.DeviceIdType.MESH))
                rdmas.append(pltpu.make_async_remote_copy(
                    src_ref=lbuf.at[hs], dst_ref=lr.at[j, hs],
                    send_sem=zs_l.at[g, j], recv_sem=zr_l.at[g, j],
                    device_id=peer, device_id_type=pl.DeviceIdType.MESH))
            return rdmas

        start_dma(0, 0)
        for g in range(N_G):
            for c in range(N_CHUNKS):
                t = g * N_CHUNKS + c
                slot = t % 2
                if t + 1 < N_G * N_CHUNKS:
                    start_dma(t + 1, (t + 1) % 2)
                for cp in chunk_copies(t, slot):
                    cp.wait()
                for hl in range(HG):
                    h = g * HG + hl
                    kb = kbuf[slot, hl].astype(jnp.bfloat16)
                    qb = qbuf[:, h, :]
                    s = lax.dot_general(
                        qb, kb, (((1,), (1,)), ((), ())),
                        preferred_element_type=jnp.float32)
                    p = jnp.exp(s)
                    lsum = jnp.sum(p, axis=1, keepdims=True)
                    lbuf[h] = lbuf[h] + jnp.broadcast_to(lsum, (SQ, LW))
                    vb = vbuf[slot, hl].astype(jnp.bfloat16)
                    pv = lax.dot_general(
                        p.astype(jnp.bfloat16), vb,
                        (((1,), (0,)), ((), ())),
                        preferred_element_type=jnp.float32)
                    acc[h] = acc[h] + pv
            hs = pl.ds(g * HG, HG)
            accsend[hs] = acc[hs].astype(jnp.bfloat16)
            for r in z_rdmas(g):
                r.start()

        for g in range(N_G):
            for r in z_rdmas(g):
                r.wait()
        acc[...] = (acc[...]
                    + accr[0].astype(jnp.float32)
                    + accr[1].astype(jnp.float32)
                    + accr[2].astype(jnp.float32))
        lbuf[...] = lbuf[...] + lr[0] + lr[1] + lr[2]

        for h in range(H):
            linv = jnp.broadcast_to(lbuf[h][:, 0:1], (SQ, D))
            obuf[my_b, :, h, :] = (acc[h] / linv).astype(jnp.bfloat16)

        peers = ((1 - my_x, my_y, my_z), (my_x, 1 - my_y, my_z),
                 (1 - my_x, 1 - my_y, my_z))
        grdmas = []
        for i, peer in enumerate(peers):
            grdmas.append(pltpu.make_async_remote_copy(
                src_ref=obuf.at[my_b], dst_ref=obuf.at[my_b],
                send_sem=gsend_sems.at[i], recv_sem=grecv_sems.at[i],
                device_id=peer, device_id_type=pl.DeviceIdType.MESH))
        for r in grdmas:
            r.start()
        for r in grdmas:
            r.wait()

        out_ref[...] = obuf[...].astype(jnp.float32)

    return pl.pallas_call(
        body,
        out_shape=jax.ShapeDtypeStruct((B, SQ, H, D), jnp.float32),
        in_specs=[
            pl.BlockSpec(memory_space=pltpu.VMEM),
            pl.BlockSpec(memory_space=pl.ANY),
            pl.BlockSpec(memory_space=pl.ANY),
        ],
        out_specs=pl.BlockSpec(memory_space=pltpu.VMEM),
        scratch_shapes=[
            pltpu.VMEM((SQ, H, D), jnp.bfloat16),
            pltpu.VMEM((2, HG, CK, D), jnp.float32),
            pltpu.VMEM((2, HG, CK, D), jnp.float32),
            pltpu.VMEM((H, SQ, D), jnp.float32),
            pltpu.VMEM((H, SQ, LW), jnp.float32),
            pltpu.VMEM((H, SQ, D), jnp.bfloat16),
            pltpu.VMEM((3, H, SQ, D), jnp.bfloat16),
            pltpu.VMEM((3, H, SQ, LW), jnp.float32),
            pltpu.VMEM((B, SQ, H, D), jnp.bfloat16),
            pltpu.SemaphoreType.DMA((2, HG)),
            pltpu.SemaphoreType.DMA((2, HG)),
            pltpu.SemaphoreType.DMA((N_G, 3)),
            pltpu.SemaphoreType.DMA((N_G, 3)),
            pltpu.SemaphoreType.DMA((N_G, 3)),
            pltpu.SemaphoreType.DMA((N_G, 3)),
            pltpu.SemaphoreType.DMA((3,)),
            pltpu.SemaphoreType.DMA((3,)),
        ],
        compiler_params=pltpu.CompilerParams(collective_id=0),
    )(Q, K, V)


# device time: 29082 ns/iter; 7.5414x vs baseline; 1.0229x over previous
import jax
import jax.numpy as jnp
from jax import lax
from jax.experimental import pallas as pl
from jax.experimental.pallas import tpu as pltpu

B, SQ, H, D = 4, 32, 8, 128
KV_PER_SHARD = 4096
CK = 1024
N_CHUNKS = KV_PER_SHARD // CK
HG = 4
N_G = H // HG
LW = 8
SCALE = D ** -0.5


def kernel(Q, K, V):
    def body(q_ref, k_ref, v_ref, out_ref,
             qbuf, kbuf, vbuf, acc, lbuf, accsend, accr, lr, obuf,
             ksem, vsem, zs_acc, zr_acc, zs_l, zr_l,
             gsend_sems, grecv_sems):
        my_x = lax.axis_index("x")
        my_y = lax.axis_index("y")
        my_z = lax.axis_index("z")
        my_b = 2 * my_x + my_y

        for dz in (1, 2, 3):
            pl.semaphore_signal(
                pltpu.get_barrier_semaphore(), inc=1,
                device_id=(my_x, my_y, jnp.bitwise_xor(my_z, dz)),
                device_id_type=pl.DeviceIdType.MESH)
        for peer in ((1 - my_x, my_y, my_z), (my_x, 1 - my_y, my_z),
                     (1 - my_x, 1 - my_y, my_z)):
            pl.semaphore_signal(
                pltpu.get_barrier_semaphore(), inc=1, device_id=peer,
                device_id_type=pl.DeviceIdType.MESH)

        qbuf[...] = (q_ref[my_b] * SCALE).astype(jnp.bfloat16)
        acc[...] = jnp.zeros_like(acc)
        lbuf[...] = jnp.zeros_like(lbuf)

        def chunk_copies(t, slot):
            g, c = divmod(t, N_CHUNKS)
            copies = []
            for hl in range(HG):
                h = g * HG + hl
                copies.append(pltpu.make_async_copy(
                    k_ref.at[my_b, pl.ds(c * CK, CK), h, :],
                    kbuf.at[slot, hl], ksem.at[slot, hl]))
                copies.append(pltpu.make_async_copy(
                    v_ref.at[my_b, pl.ds(c * CK, CK), h, :],
                    vbuf.at[slot, hl], vsem.at[slot, hl]))
            return copies

        def start_dma(t, slot):
            for cp in chunk_copies(t, slot):
                cp.start()

        def z_rdmas(g):
            hs = pl.ds(g * HG, HG)
            rdmas = []
            for j, dz in enumerate((1, 2, 3)):
                peer = (my_x, my_y, jnp.bitwise_xor(my_z, dz))
                rdmas.append(pltpu.make_async_remote_copy(
                    src_ref=accsend.at[hs], dst_ref=accr.at[j, hs],
                    send_sem=zs_acc.at[g, j], recv_sem=zr_acc.at[g, j],
                    device_id=peer, device_id_type=pl.DeviceIdType.MESH))
                rdmas.append(pltpu.make_async_remote_copy(
                    src_ref=lbuf.at[hs], dst_ref=lr.at[j, hs],
                    send_sem=zs_l.at[g, j], recv_sem=zr_l.at[g, j],
                    device_id=peer, device_id_type=pl.DeviceIdType.MESH))
            return rdmas

        start_dma(0, 0)
        for g in range(N_G):
            for c in range(N_CHUNKS):
                t = g * N_CHUNKS + c
                slot = t % 2
                if t + 1 < N_G * N_CHUNKS:
                    start_dma(t + 1, (t + 1) % 2)
                for cp in chunk_copies(t, slot):
                    cp.wait()
                for hl in range(HG):
                    h = g * HG + hl
                    kb = kbuf[slot, hl].astype(jnp.bfloat16)
                    qb = qbuf[:, h, :]
                    s = lax.dot_general(
                        qb, kb, (((1,), (1,)), ((), ())),
                        preferred_element_type=jnp.float32)
                    p = jnp.exp(s)
                    lsum = jnp.sum(p, axis=1, keepdims=True)
                    lbuf[h] = lbuf[h] + jnp.broadcast_to(lsum, (SQ, LW))
                    vb = vbuf[slot, hl].astype(jnp.bfloat16)
                    pv = lax.dot_general(
                        p.astype(jnp.bfloat16), vb,
                        (((1,), (0,)), ((), ())),
                        preferred_element_type=jnp.float32)
                    acc[h] = acc[h] + pv
            hs = pl.ds(g * HG, HG)
            accsend[hs] = acc[hs].astype(jnp.bfloat16)
            if g == 0:
                pl.semaphore_wait(pltpu.get_barrier_semaphore(), 6)
            for r in z_rdmas(g):
                r.start()

        xy_peers = ((1 - my_x, my_y, my_z), (my_x, 1 - my_y, my_z),
                    (1 - my_x, 1 - my_y, my_z))

        def gather_rdmas(g):
            hs = pl.ds(g * HG, HG)
            return [pltpu.make_async_remote_copy(
                src_ref=obuf.at[my_b, :, hs, :],
                dst_ref=obuf.at[my_b, :, hs, :],
                send_sem=gsend_sems.at[g, i], recv_sem=grecv_sems.at[g, i],
                device_id=peer, device_id_type=pl.DeviceIdType.MESH)
                for i, peer in enumerate(xy_peers)]

        for g in range(N_G):
            for r in z_rdmas(g):
                r.wait()
            hs = pl.ds(g * HG, HG)
            acc[hs] = (acc[hs]
                       + accr[0, hs].astype(jnp.float32)
                       + accr[1, hs].astype(jnp.float32)
                       + accr[2, hs].astype(jnp.float32))
            lbuf[hs] = lbuf[hs] + lr[0, hs] + lr[1, hs] + lr[2, hs]
            for hl in range(HG):
                h = g * HG + hl
                linv = jnp.broadcast_to(lbuf[h][:, 0:1], (SQ, D))
                obuf[my_b, :, h, :] = (acc[h] / linv).astype(jnp.bfloat16)
            for r in gather_rdmas(g):
                r.start()
        for g in range(N_G):
            for r in gather_rdmas(g):
                r.wait()

        out_ref[...] = obuf[...].astype(jnp.float32)

    return pl.pallas_call(
        body,
        out_shape=jax.ShapeDtypeStruct((B, SQ, H, D), jnp.float32),
        in_specs=[
            pl.BlockSpec(memory_space=pltpu.VMEM),
            pl.BlockSpec(memory_space=pl.ANY),
            pl.BlockSpec(memory_space=pl.ANY),
        ],
        out_specs=pl.BlockSpec(memory_space=pltpu.VMEM),
        scratch_shapes=[
            pltpu.VMEM((SQ, H, D), jnp.bfloat16),
            pltpu.VMEM((2, HG, CK, D), jnp.float32),
            pltpu.VMEM((2, HG, CK, D), jnp.float32),
            pltpu.VMEM((H, SQ, D), jnp.float32),
            pltpu.VMEM((H, SQ, LW), jnp.float32),
            pltpu.VMEM((H, SQ, D), jnp.bfloat16),
            pltpu.VMEM((3, H, SQ, D), jnp.bfloat16),
            pltpu.VMEM((3, H, SQ, LW), jnp.float32),
            pltpu.VMEM((B, SQ, H, D), jnp.bfloat16),
            pltpu.SemaphoreType.DMA((2, HG)),
            pltpu.SemaphoreType.DMA((2, HG)),
            pltpu.SemaphoreType.DMA((N_G, 3)),
            pltpu.SemaphoreType.DMA((N_G, 3)),
            pltpu.SemaphoreType.DMA((N_G, 3)),
            pltpu.SemaphoreType.DMA((N_G, 3)),
            pltpu.SemaphoreType.DMA((N_G, 3)),
            pltpu.SemaphoreType.DMA((N_G, 3)),
        ],
        compiler_params=pltpu.CompilerParams(collective_id=0),
    )(Q, K, V)
